# baseline (speedup 1.0000x reference)
import jax
import jax.numpy as jnp
from jax.experimental import pallas as pl

N = 10000


def _softmax_kernel(l_ref, o_ref):
    l = l_ref[...]
    m = jnp.max(l, axis=1, keepdims=True)
    e = jnp.exp(l - m)
    o_ref[...] = e / jnp.sum(e, axis=1, keepdims=True)


def kernel(x, edge_index, W0, b0, W1, b1, W2, b2):
    src = edge_index[0]
    dst = edge_index[1]
    loop = jnp.arange(N, dtype=src.dtype)
    src = jnp.concatenate([src, loop])
    dst = jnp.concatenate([dst, loop])
    deg = jax.ops.segment_sum(jnp.ones_like(dst, dtype=jnp.float32), dst, num_segments=N)
    dinv = jax.lax.rsqrt(jnp.maximum(deg, 1.0))
    norm = dinv[src] * dinv[dst]

    def conv(h, W, b):
        h = h @ W
        msg = h[src] * norm[:, None]
        return jax.ops.segment_sum(msg, dst, num_segments=N) + b

    h = jax.nn.relu(conv(x, W0, b0))
    h = jax.nn.relu(conv(h, W1, b1))
    logits = conv(h, W2, b2)
    return pl.pallas_call(
        _softmax_kernel,
        out_shape=jax.ShapeDtypeStruct((N, logits.shape[1]), jnp.float32),
        grid=(10,),
        in_specs=[pl.BlockSpec((N // 10, logits.shape[1]), lambda i: (i, 0))],
        out_specs=pl.BlockSpec((N // 10, logits.shape[1]), lambda i: (i, 0)),
    )(logits)


# speedup vs baseline: 7.1953x; 7.1953x over previous
import functools

import jax
import jax.numpy as jnp
from jax import lax
from jax.experimental import pallas as pl
from jax.experimental.pallas import tpu as pltpu
from jax.experimental.pallas import tpu_sc as plsc

N = 10000
E = 160000
D = 256
H = 128
POUT = 128
NCLS = 40

NC = 2
NS = 16
CH = 128
EP = 163840
NCHUNKS = EP // CH
NCHT = NCHUNKS // NS
NCHT2 = NCHUNKS // (NC * NS)
RP = 10240
RPT = RP // NS

_MESH = plsc.VectorSubcoreMesh(
    core_axis_name="c", subcore_axis_name="s", num_cores=NC, num_subcores=NS)


def _f32(*shape):
    return jax.ShapeDtypeStruct(shape, jnp.float32)


def _sc_deg_body(dst2d, zeros, ones, out, dst_v, ones_v, acc):
    c = lax.axis_index("c")
    s = lax.axis_index("s")
    pltpu.sync_copy(zeros, acc.at[pl.ds(s * RPT, RPT)])
    pltpu.sync_copy(ones, ones_v)
    base = (c * NS + s) * NCHT2
    pltpu.sync_copy(dst2d.at[pl.ds(base, NCHT2)], dst_v)
    plsc.subcore_barrier()

    def body(k, carry):
        pltpu.sync_copy(ones_v, acc.at[dst_v.at[k]], add=True)
        return carry

    lax.fori_loop(0, NCHT2, body, 0)
    plsc.subcore_barrier()
    pltpu.sync_copy(acc.at[pl.ds(s * RPT, RPT)],
                    out.at[c, pl.ds(s * RPT, RPT)])


_sc_deg = pl.kernel(
    _sc_deg_body,
    out_type=_f32(NC, RP, H),
    mesh=_MESH,
    scratch_types=[
        pltpu.VMEM((NCHT2, CH), jnp.int32),
        pltpu.VMEM((CH, H), jnp.float32),
        pltpu.VMEM_SHARED((RP, H), jnp.float32),
    ],
)


def _sc_agg_feat_body(t0, t1, src2d, dst2d, zeros, out,
                      src_v, dst_v, rows, acc, sem0, sem1):
    c = lax.axis_index("c")
    s = lax.axis_index("s")
    pltpu.sync_copy(zeros, acc.at[pl.ds(s * RPT, RPT)])

    def gstart(k, buf, sem):
        @pl.when(c == 0)
        def _():
            pltpu.async_copy(t0.at[src_v.at[k]], buf, sem)

        @pl.when(c == 1)
        def _():
            pltpu.async_copy(t1.at[src_v.at[k]], buf, sem)

    def gwait(buf, sem):
        pltpu.make_async_copy(t0.at[src_v.at[0]], buf, sem).wait()

    def scat(k, buf):
        pltpu.sync_copy(buf, acc.at[dst_v.at[k]], add=True)

    for p in range(NCHT // NCHT2):
        base = s * NCHT + p * NCHT2
        pltpu.sync_copy(src2d.at[pl.ds(base, NCHT2)], src_v)
        pltpu.sync_copy(dst2d.at[pl.ds(base, NCHT2)], dst_v)
        if p == 0:
            plsc.subcore_barrier()
        gstart(0, rows.at[0], sem0)

        def body(j, carry):
            k0 = 2 * j
            gstart(k0 + 1, rows.at[1], sem1)
            gwait(rows.at[0], sem0)
            scat(k0, rows.at[0])

            @pl.when(j < NCHT2 // 2 - 1)
            def _():
                gstart(k0 + 2, rows.at[0], sem0)

            gwait(rows.at[1], sem1)
            scat(k0 + 1, rows.at[1])
            return carry

        lax.fori_loop(0, NCHT2 // 2, body, 0)
    plsc.subcore_barrier()
    pltpu.sync_copy(acc.at[pl.ds(s * RPT, RPT)],
                    out.at[c, pl.ds(s * RPT, RPT)])


_sc_agg_feat = pl.kernel(
    _sc_agg_feat_body,
    out_type=_f32(NC, RP, H),
    mesh=_MESH,
    scratch_types=[
        pltpu.VMEM((NCHT2, CH), jnp.int32),
        pltpu.VMEM((NCHT2, CH), jnp.int32),
        pltpu.VMEM((2, CH, H), jnp.float32),
        pltpu.VMEM_SHARED((RP, H), jnp.float32),
        pltpu.SemaphoreType.DMA,
        pltpu.SemaphoreType.DMA,
    ],
)


def _sc_agg_edge_body(t, src2d, dst2d, zeros, out,
                      src_v, dst_v, rows, acc, sem0, sem1):
    c = lax.axis_index("c")
    s = lax.axis_index("s")
    pltpu.sync_copy(zeros, acc.at[pl.ds(s * RPT, RPT)])
    base = (c * NS + s) * NCHT2
    pltpu.sync_copy(src2d.at[pl.ds(base, NCHT2)], src_v)
    pltpu.sync_copy(dst2d.at[pl.ds(base, NCHT2)], dst_v)
    plsc.subcore_barrier()

    def gstart(k, buf, sem):
        pltpu.async_copy(t.at[src_v.at[k]], buf, sem)

    def gwait(buf, sem):
        pltpu.make_async_copy(t.at[src_v.at[0]], buf, sem).wait()

    def scat(k, buf):
        pltpu.sync_copy(buf, acc.at[dst_v.at[k]], add=True)

    gstart(0, rows.at[0], sem0)

    def body(j, carry):
        k0 = 2 * j
        gstart(k0 + 1, rows.at[1], sem1)
        gwait(rows.at[0], sem0)
        scat(k0, rows.at[0])

        @pl.when(j < NCHT2 // 2 - 1)
        def _():
            gstart(k0 + 2, rows.at[0], sem0)

        gwait(rows.at[1], sem1)
        scat(k0 + 1, rows.at[1])
        return carry

    lax.fori_loop(0, NCHT2 // 2, body, 0)
    plsc.subcore_barrier()
    pltpu.sync_copy(acc.at[pl.ds(s * RPT, RPT)],
                    out.at[c, pl.ds(s * RPT, RPT)])


_sc_agg_edge = pl.kernel(
    _sc_agg_edge_body,
    out_type=_f32(NC, RP, POUT),
    mesh=_MESH,
    scratch_types=[
        pltpu.VMEM((NCHT2, CH), jnp.int32),
        pltpu.VMEM((NCHT2, CH), jnp.int32),
        pltpu.VMEM((2, CH, POUT), jnp.float32),
        pltpu.VMEM_SHARED((RP, POUT), jnp.float32),
        pltpu.SemaphoreType.DMA,
        pltpu.SemaphoreType.DMA,
    ],
)


BR = 1000
GRID = (N // BR,)


def _tc1_body(deg_ref, x_ref, w_ref, ha_ref, hb_ref, dinv_ref):
    deg = deg_ref[0, :, 0:1] + deg_ref[1, :, 0:1] + 1.0
    dinv = lax.rsqrt(jnp.maximum(deg, 1.0))
    h = jnp.dot(x_ref[...], w_ref[...], preferred_element_type=jnp.float32)
    h = h * dinv
    ha_ref[...] = h[:, :H]
    hb_ref[...] = h[:, H:]
    dinv_ref[...] = dinv


def _tc_mid_body(agg_ref, ha_ref, hb_ref, dinv_ref, b_ref, w_ref, *outs):
    dinv = dinv_ref[...]
    left = agg_ref[0] + ha_ref[...]
    right = agg_ref[1] + hb_ref[...]
    pre = jnp.concatenate([left, right], axis=1) * dinv + b_ref[...]
    h = jnp.maximum(pre, 0.0)
    hw = jnp.dot(h, w_ref[...], preferred_element_type=jnp.float32) * dinv
    if len(outs) == 2:
        outs[0][...] = hw[:, :H]
        outs[1][...] = hw[:, H:]
    else:
        outs[0][...] = hw


def _tc4_body(agg_ref, h2_ref, dinv_ref, b_ref, o_ref):
    l = (agg_ref[0] + agg_ref[1] + h2_ref[...]) * dinv_ref[...] + b_ref[...]
    l40 = l[:, :NCLS]
    m = jnp.max(l40, axis=1, keepdims=True)
    e = jnp.exp(l40 - m)
    o_ref[...] = e / jnp.sum(e, axis=1, keepdims=True)


def _rows_spec(w):
    return pl.BlockSpec((BR, w), lambda i: (i, 0))


def _pair_spec(w):
    return pl.BlockSpec((NC, BR, w), lambda i: (0, i, 0))


def _full_spec(a, b):
    return pl.BlockSpec((a, b), lambda i: (0, 0))


_tc1 = pl.pallas_call(
    _tc1_body,
    grid=GRID,
    in_specs=[_pair_spec(H), _rows_spec(D), _full_spec(D, D)],
    out_specs=[_rows_spec(H), _rows_spec(H), _rows_spec(1)],
    out_shape=[_f32(N, H), _f32(N, H), _f32(N, 1)],
)

_tc2 = pl.pallas_call(
    _tc_mid_body,
    grid=GRID,
    in_specs=[_pair_spec(H), _rows_spec(H), _rows_spec(H), _rows_spec(1),
              _full_spec(1, D), _full_spec(D, D)],
    out_specs=[_rows_spec(H), _rows_spec(H)],
    out_shape=[_f32(N, H), _f32(N, H)],
)

_tc3 = pl.pallas_call(
    _tc_mid_body,
    grid=GRID,
    in_specs=[_pair_spec(H), _rows_spec(H), _rows_spec(H), _rows_spec(1),
              _full_spec(1, D), _full_spec(D, POUT)],
    out_specs=[_rows_spec(POUT)],
    out_shape=[_f32(N, POUT)],
)

_tc4 = pl.pallas_call(
    _tc4_body,
    grid=GRID,
    in_specs=[_pair_spec(POUT), _rows_spec(POUT), _rows_spec(1),
              _full_spec(1, POUT)],
    out_specs=_rows_spec(NCLS),
    out_shape=_f32(N, NCLS),
)


@jax.jit
def kernel(x, edge_index, W0, b0, W1, b1, W2, b2):
    src = edge_index[0]
    dst = edge_index[1]
    pad = EP - E
    src2d = jnp.concatenate(
        [src, jnp.zeros((pad,), jnp.int32)]).reshape(NCHUNKS, CH)
    dst2d = jnp.concatenate(
        [dst, jnp.full((pad,), N, jnp.int32)]).reshape(NCHUNKS, CH)

    zeros_h = jnp.zeros((RPT, H), jnp.float32)
    zeros_p = jnp.zeros((RPT, POUT), jnp.float32)
    ones_h = jnp.ones((CH, H), jnp.float32)
    W2p = jnp.zeros((D, POUT), jnp.float32).at[:, :NCLS].set(W2)
    b0r = b0.reshape(1, D)
    b1r = b1.reshape(1, D)
    b2p = jnp.zeros((1, POUT), jnp.float32).at[0, :NCLS].set(b2)

    degp = _sc_deg(dst2d, zeros_h, ones_h)
    h0a, h0b, dinv = _tc1(degp, x, W0)
    agg0 = _sc_agg_feat(h0a, h0b, src2d, dst2d, zeros_h)
    h1a, h1b = _tc2(agg0, h0a, h0b, dinv, b0r, W1)
    agg1 = _sc_agg_feat(h1a, h1b, src2d, dst2d, zeros_h)
    (h2,) = _tc3(agg1, h1a, h1b, dinv, b1r, W2p)
    agg2 = _sc_agg_edge(h2, src2d, dst2d, zeros_p)
    return _tc4(agg2, h2, dinv, b2p)
